# Initial kernel scaffold; baseline (speedup 1.0000x reference)
import jax
import jax.numpy as jnp
from jax.experimental import pallas as pl


def kernel(x, edge_index, batch, W1, a1s, a1d, b1, W2, a2s, a2d, b2, W3, a3s, a3d, b3, fcW1, fcb1, fcW2, fcb2):
    raise NotImplementedError("write your pallas kernel here")


# speedup vs baseline: 5.2810x; 5.2810x over previous
import functools

import jax
import jax.numpy as jnp
from jax.experimental import pallas as pl

N_TILE = 1000
E_TILE = 2000


def _elu(v):
    return jnp.where(v > 0, v, jnp.exp(jnp.minimum(v, 0.0)) - 1.0)


def _node_kernel(x_ref, w_ref, as_ref, ad_ref, bprev_ref,
                 h_ref, als_ref, ald_ref, *, pre_act):
    x = x_ref[...] + bprev_ref[...]
    if pre_act:
        x = _elu(x)
    h = jnp.dot(x, w_ref[...], preferred_element_type=jnp.float32)
    h_ref[...] = h
    als_ref[...] = jnp.dot(h, as_ref[...], preferred_element_type=jnp.float32)
    ald_ref[...] = jnp.dot(h, ad_ref[...], preferred_element_type=jnp.float32)


def _node_stage(x, W, A_s, A_d, b_prev, pre_act, H):
    n, fin = x.shape
    hc = W.shape[1]
    grid = n // N_TILE
    out_shapes = (
        jax.ShapeDtypeStruct((n, hc), jnp.float32),
        jax.ShapeDtypeStruct((n, H), jnp.float32),
        jax.ShapeDtypeStruct((n, H), jnp.float32),
    )
    return pl.pallas_call(
        functools.partial(_node_kernel, pre_act=pre_act),
        grid=(grid,),
        in_specs=[
            pl.BlockSpec((N_TILE, fin), lambda i: (i, 0)),
            pl.BlockSpec((fin, hc), lambda i: (0, 0)),
            pl.BlockSpec((hc, H), lambda i: (0, 0)),
            pl.BlockSpec((hc, H), lambda i: (0, 0)),
            pl.BlockSpec((1, fin), lambda i: (0, 0)),
        ],
        out_specs=(
            pl.BlockSpec((N_TILE, hc), lambda i: (i, 0)),
            pl.BlockSpec((N_TILE, H), lambda i: (i, 0)),
            pl.BlockSpec((N_TILE, H), lambda i: (i, 0)),
        ),
        out_shape=out_shapes,
    )(x, W, A_s, A_d, b_prev.reshape(1, fin))


def _edge_logit_kernel(es_ref, ed_ref, out_ref):
    e = es_ref[...] + ed_ref[...]
    out_ref[...] = jnp.where(e > 0, e, 0.2 * e)


def _edge_exp_kernel(e_ref, m_ref, out_ref):
    out_ref[...] = jnp.exp(e_ref[...] - m_ref[...])


def _edge_msg_kernel(h_ref, eexp_ref, ssum_ref, rep_ref, out_ref):
    w = eexp_ref[...] / (ssum_ref[...] + 1e-16)
    out_ref[...] = h_ref[...] * jnp.dot(
        w, rep_ref[...], preferred_element_type=jnp.float32)


def _edge_ew(kern, out_cols, *arrs):
    e = arrs[0].shape[0]
    grid = e // E_TILE
    return pl.pallas_call(
        kern,
        grid=(grid,),
        in_specs=[pl.BlockSpec((E_TILE, a.shape[1]), lambda i: (i, 0))
                  for a in arrs],
        out_specs=pl.BlockSpec((E_TILE, out_cols), lambda i: (i, 0)),
        out_shape=jax.ShapeDtypeStruct((e, out_cols), jnp.float32),
    )(*arrs)


def _edge_msg(h_src, e_exp, ssum_dst, rep):
    e = h_src.shape[0]
    hc = h_src.shape[1]
    grid = e // E_TILE
    return pl.pallas_call(
        _edge_msg_kernel,
        grid=(grid,),
        in_specs=[
            pl.BlockSpec((E_TILE, hc), lambda i: (i, 0)),
            pl.BlockSpec((E_TILE, rep.shape[0]), lambda i: (i, 0)),
            pl.BlockSpec((E_TILE, rep.shape[0]), lambda i: (i, 0)),
            pl.BlockSpec((rep.shape[0], hc), lambda i: (0, 0)),
        ],
        out_specs=pl.BlockSpec((E_TILE, hc), lambda i: (i, 0)),
        out_shape=jax.ShapeDtypeStruct((e, hc), jnp.float32),
    )(h_src, e_exp, ssum_dst, rep)


def _gat_layer(x, src, dst, W, a_s, a_d, b_prev, pre_act):
    H, C = a_s.shape
    n = x.shape[0]
    eye = jnp.eye(H, dtype=jnp.float32)
    A_s = (eye[:, None, :] * a_s[:, :, None]).reshape(H * C, H)
    A_d = (eye[:, None, :] * a_d[:, :, None]).reshape(H * C, H)
    h, al_s, al_d = _node_stage(x, W, A_s, A_d, b_prev, pre_act, H)
    e_raw = _edge_ew(_edge_logit_kernel, H, al_s[src], al_d[dst])
    m = jax.ops.segment_max(e_raw, dst, num_segments=n)
    e_exp = _edge_ew(_edge_exp_kernel, H, e_raw, m[dst])
    ssum = jax.ops.segment_sum(e_exp, dst, num_segments=n)
    rep = jnp.kron(jnp.eye(H, dtype=jnp.float32),
                   jnp.ones((1, C), dtype=jnp.float32))
    msg = _edge_msg(h[src], e_exp, ssum[dst], rep)
    return jax.ops.segment_sum(msg, dst, num_segments=n)


def _pool_kernel(h_ref, b_ref, batch_ref, w1_ref, b1_ref, w2_ref, b2_ref,
                 out_ref, sums_ref, cnt_ref, *, grid, n_graphs):
    i = pl.program_id(0)

    @pl.when(i == 0)
    def _init():
        sums_ref[...] = jnp.zeros_like(sums_ref)
        cnt_ref[...] = jnp.zeros_like(cnt_ref)
        out_ref[...] = jnp.zeros_like(out_ref)

    h = _elu(h_ref[...] + b_ref[...])
    gids = jax.lax.broadcasted_iota(jnp.int32, (N_TILE, n_graphs), 1)
    onehot = (batch_ref[...] == gids).astype(jnp.float32)
    sums_ref[...] += jnp.dot(onehot.T, h, preferred_element_type=jnp.float32)
    cnt_ref[...] += jnp.sum(onehot.T, axis=1, keepdims=True)

    @pl.when(i == grid - 1)
    def _head():
        pooled = sums_ref[...] / jnp.maximum(cnt_ref[...], 1.0)
        z = _elu(jnp.dot(pooled, w1_ref[...],
                         preferred_element_type=jnp.float32) + b1_ref[...])
        out_ref[...] = jnp.dot(z, w2_ref[...],
                               preferred_element_type=jnp.float32) + b2_ref[...]


def _pool_head(h3, b3, batch, fcW1, fcb1, fcW2, fcb2, n_graphs):
    n, hc = h3.shape
    grid = n // N_TILE
    hid = fcW1.shape[1]
    out, _, _ = pl.pallas_call(
        functools.partial(_pool_kernel, grid=grid, n_graphs=n_graphs),
        grid=(grid,),
        in_specs=[
            pl.BlockSpec((N_TILE, hc), lambda i: (i, 0)),
            pl.BlockSpec((1, hc), lambda i: (0, 0)),
            pl.BlockSpec((N_TILE, 1), lambda i: (i, 0)),
            pl.BlockSpec((hc, hid), lambda i: (0, 0)),
            pl.BlockSpec((1, hid), lambda i: (0, 0)),
            pl.BlockSpec((hid, 1), lambda i: (0, 0)),
            pl.BlockSpec((1, 1), lambda i: (0, 0)),
        ],
        out_specs=(
            pl.BlockSpec((n_graphs, 1), lambda i: (0, 0)),
            pl.BlockSpec((n_graphs, hc), lambda i: (0, 0)),
            pl.BlockSpec((n_graphs, 1), lambda i: (0, 0)),
        ),
        out_shape=(
            jax.ShapeDtypeStruct((n_graphs, 1), jnp.float32),
            jax.ShapeDtypeStruct((n_graphs, hc), jnp.float32),
            jax.ShapeDtypeStruct((n_graphs, 1), jnp.float32),
        ),
    )(h3, b3.reshape(1, hc), batch.reshape(n, 1), fcW1,
      fcb1.reshape(1, hid), fcW2, fcb2.reshape(1, 1))
    return out


def kernel(x, edge_index, batch, W1, a1s, a1d, b1, W2, a2s, a2d, b2,
           W3, a3s, a3d, b3, fcW1, fcb1, fcW2, fcb2):
    n = x.shape[0]
    loop = jnp.arange(n, dtype=edge_index.dtype)
    src = jnp.concatenate([edge_index[0], loop])
    dst = jnp.concatenate([edge_index[1], loop])
    zb = jnp.zeros((x.shape[1],), dtype=jnp.float32)
    h1 = _gat_layer(x, src, dst, W1, a1s, a1d, zb, pre_act=False)
    h2 = _gat_layer(h1, src, dst, W2, a2s, a2d, b1, pre_act=True)
    h3 = _gat_layer(h2, src, dst, W3, a3s, a3d, b2, pre_act=True)
    return _pool_head(h3, b3, batch, fcW1, fcb1, fcW2, fcb2, 64)
